# baseline (speedup 1.0000x reference)
import jax
import jax.numpy as jnp
from jax.experimental import pallas as pl


def _copy3(x_ref, e_ref, u_ref, xo_ref, eo_ref, uo_ref):
    xo_ref[...] = x_ref[...]
    eo_ref[...] = e_ref[...]
    uo_ref[...] = u_ref[...]


def kernel(x, edge_index, edge_attr, u, batch):
    del edge_index, batch
    e_shape = edge_attr.shape
    e2 = edge_attr.reshape(e_shape[0] * e_shape[1] // 128, 128)
    grid = 5
    xb = x.shape[0] // grid
    eb = e2.shape[0] // grid
    outs = pl.pallas_call(
        _copy3,
        grid=(grid,),
        in_specs=[
            pl.BlockSpec((xb, x.shape[1]), lambda i: (i, 0)),
            pl.BlockSpec((eb, 128), lambda i: (i, 0)),
            pl.BlockSpec(u.shape, lambda i: (0, 0)),
        ],
        out_specs=[
            pl.BlockSpec((xb, x.shape[1]), lambda i: (i, 0)),
            pl.BlockSpec((eb, 128), lambda i: (i, 0)),
            pl.BlockSpec(u.shape, lambda i: (0, 0)),
        ],
        out_shape=[
            jax.ShapeDtypeStruct(x.shape, x.dtype),
            jax.ShapeDtypeStruct(e2.shape, e2.dtype),
            jax.ShapeDtypeStruct(u.shape, u.dtype),
        ],
    )(x, e2, u)
    return (outs[0], outs[1].reshape(e_shape), outs[2])
